# baseline (speedup 1.0000x reference)
import functools

import jax
import jax.numpy as jnp
import numpy as np
from jax.experimental import pallas as pl
from jax.experimental.pallas import tpu as pltpu
from jax.experimental.pallas import tpu_sc as plsc
from jax.sharding import Mesh, PartitionSpec as P

_VOCAB = 8192
_WIDTH = 32
_BETA = 0.25
_TOK_BLK = 256
_GATHER_WIN = 128
_GATHER_PAD = 128


def _prep_body(cb_ref, cbn_ref, cbt_ref):
    cb = cb_ref[...]
    ss = jnp.sum(cb * cb, axis=1, keepdims=True)
    r = 1.0 / jnp.maximum(jnp.sqrt(ss), 1e-12)
    cbn = cb * r
    pad = jnp.zeros((_VOCAB, _GATHER_PAD - _WIDTH), jnp.float32)
    cbn_ref[...] = jnp.concatenate([cbn, pad], axis=1)
    cbt_ref[...] = cbn.T


def _argmax_body(nblk, f_a_ref, f_b_ref, cbt_ref, idx_e_ref, idx_o_ref,
                 used_ref, stats_ref, buf_a, buf_b):
    j = pl.program_id(0)
    m_steps = nblk // 2

    @pl.when(j == 0)
    def _():
        stats_ref[0, 0] = 0.0
        stats_ref[0, 1] = 0.0

    def produce(f_ref, buf, fresh):
        f = f_ref[...]
        ss = jnp.sum(f * f, axis=1, keepdims=True)
        fn = f * (1.0 / jnp.maximum(jnp.sqrt(ss), 1e-12))
        buf[...] = jnp.dot(fn, cbt_ref[...],
                           preferred_element_type=jnp.float32)
        stats_ref[0, 1] += jnp.where(fresh, jnp.sum(fn * fn), 0.0)

    def consume(buf, idx_out_ref, fresh):
        s = buf[...]
        m = jnp.max(s, axis=1, keepdims=True)
        mask = s >= m
        idx_out_ref[0, 0, :] = jnp.argmax(s, axis=1).astype(jnp.int32)
        stats_ref[0, 0] += jnp.where(fresh, jnp.sum(m), 0.0)
        return jnp.any(mask.reshape(_TOK_BLK // 8, 8, _VOCAB), axis=0)

    produce(f_a_ref, buf_a, j < m_steps)
    mf_b = consume(buf_b, idx_o_ref, j > 0)
    produce(f_b_ref, buf_b, j < m_steps)
    mf_a = consume(buf_a, idx_e_ref, j < m_steps)

    g_b = jnp.logical_and(j > 0, mf_b)
    contrib = jnp.logical_or(mf_a, g_b).astype(jnp.int32)
    used_ref[...] = jnp.where(j == 0, contrib, used_ref[...] | contrib)


def _tc_argmax(f, cbn_t):
    n = f.shape[0]
    nblk = n // _TOK_BLK
    m_steps = nblk // 2
    body = functools.partial(_argmax_body, nblk)
    return pl.pallas_call(
        body,
        grid=(m_steps + 1,),
        in_specs=[
            pl.BlockSpec((_TOK_BLK, _WIDTH),
                         lambda j: (jnp.minimum(2 * j, nblk - 2), 0)),
            pl.BlockSpec((_TOK_BLK, _WIDTH),
                         lambda j: (jnp.minimum(2 * j + 1, nblk - 1), 0)),
            pl.BlockSpec((_WIDTH, _VOCAB), lambda j: (0, 0)),
        ],
        out_specs=[
            pl.BlockSpec((1, 1, _TOK_BLK),
                         lambda j: (jnp.minimum(j, m_steps - 1), 0, 0)),
            pl.BlockSpec((1, 1, _TOK_BLK),
                         lambda j: (jnp.maximum(j - 1, 0), 0, 0)),
            pl.BlockSpec((8, _VOCAB), lambda j: (0, 0)),
            pl.BlockSpec(memory_space=pltpu.SMEM),
        ],
        out_shape=[
            jax.ShapeDtypeStruct((m_steps, 1, _TOK_BLK), jnp.int32),
            jax.ShapeDtypeStruct((m_steps, 1, _TOK_BLK), jnp.int32),
            jax.ShapeDtypeStruct((8, _VOCAB), jnp.int32),
            jax.ShapeDtypeStruct((1, 4), jnp.float32),
        ],
        scratch_shapes=[
            pltpu.VMEM((_TOK_BLK, _VOCAB), jnp.float32),
            pltpu.VMEM((_TOK_BLK, _VOCAB), jnp.float32),
        ],
    )(f, f, cbn_t)


def _prep(cb):
    return pl.pallas_call(
        _prep_body,
        out_shape=[
            jax.ShapeDtypeStruct((_VOCAB, _GATHER_PAD), jnp.float32),
            jax.ShapeDtypeStruct((_WIDTH, _VOCAB), jnp.float32),
        ],
    )(cb)


def _sc_gather(cbn, idx2d):
    n = idx2d.shape[1]
    mesh = plsc.VectorSubcoreMesh(core_axis_name="c", subcore_axis_name="s")

    @functools.partial(
        pl.kernel,
        out_type=jax.ShapeDtypeStruct((n, _GATHER_PAD), jnp.float32),
        mesh=mesh,
    )
    def gather_kernel(cb_hbm, i_hbm, o_hbm):
        def body(i_vmem, o_vmem):
            pltpu.sync_copy(cb_hbm.at[i_vmem.at[0]], o_vmem)

        pltpu.emit_pipeline(
            body,
            grid=(n // _GATHER_WIN,),
            in_specs=[pl.BlockSpec((1, _GATHER_WIN), lambda i: (0, i))],
            out_specs=[pl.BlockSpec((_GATHER_WIN, _GATHER_PAD),
                                    lambda i: (i, 0))],
            core_axis_name="s",
            dimension_semantics=(pltpu.PARALLEL,),
        )(i_hbm, o_hbm)

    return gather_kernel(cbn, idx2d)


def _shard_fn(n_total, axis, features, codebook_weight):
    b, l, c = features.shape
    f = features.reshape(-1, c)
    cbn, cbn_t = _prep(codebook_weight)
    idx_e, idx_o, used, stats = _tc_argmax(f, cbn_t)
    idx = jnp.stack([idx_e[:, 0, :], idx_o[:, 0, :]], axis=1)
    f_hat = _sc_gather(cbn, idx.reshape(1, -1))[:, :_WIDTH]
    s_sum = stats[0, 0]
    nf2_sum = stats[0, 1]
    if axis is not None:
        s_sum = jax.lax.psum(s_sum, axis)
        nf2_sum = jax.lax.psum(nf2_sum, axis)
        used = jax.lax.pmax(used, axis)
    n = jnp.float32(n_total)
    vq_loss = (1.0 + _BETA) * (nf2_sum - 2.0 * s_sum + n) / (n * _WIDTH)
    used_row = jnp.max(used, axis=0).astype(jnp.float32)
    vocab_usage = 100.0 * jnp.sum(used_row) / _VOCAB
    return (f_hat.reshape(b, l, c), vq_loss, jnp.float32(0.0), vocab_usage)


def kernel(features, codebook_weight):
    n_total = features.shape[0] * features.shape[1]
    tpu_devs = [d for d in jax.devices() if d.platform == "tpu"]
    if len(tpu_devs) >= 2 and features.shape[0] % 2 == 0:
        mesh = Mesh(np.array(tpu_devs[:2]), ("x",))
        fn = functools.partial(_shard_fn, n_total, "x")
        return jax.shard_map(
            fn,
            mesh=mesh,
            in_specs=(P("x", None, None), P(None, None)),
            out_specs=(P("x", None, None), P(), P(), P()),
            check_vma=False,
        )(features, codebook_weight)
    return _shard_fn(n_total, None, features, codebook_weight)


# speedup vs baseline: 1.4573x; 1.4573x over previous
import functools

import jax
import jax.numpy as jnp
import numpy as np
from jax.experimental import pallas as pl
from jax.experimental.pallas import tpu as pltpu
from jax.experimental.pallas import tpu_sc as plsc
from jax.sharding import Mesh, PartitionSpec as P

_VOCAB = 8192
_WIDTH = 32
_BETA = 0.25
_TOK_BLK = 256
_GATHER_WIN = 128
_GATHER_PAD = 128


def _prep_body(cb_ref, cbn_ref, cbt_ref):
    cb = cb_ref[...]
    ss = jnp.sum(cb * cb, axis=1, keepdims=True)
    r = 1.0 / jnp.maximum(jnp.sqrt(ss), 1e-12)
    cbn = cb * r
    pad = jnp.zeros((_VOCAB, _GATHER_PAD - _WIDTH), jnp.float32)
    cbn_ref[...] = jnp.concatenate([cbn, pad], axis=1)
    cbt_ref[...] = cbn.T


def _argmax_body(nblk, f_a_ref, f_b_ref, cbt_ref, idx_e_ref, idx_o_ref,
                 used_ref, stats_ref, buf_a, buf_b):
    j = pl.program_id(0)
    m_steps = nblk // 2

    @pl.when(j == 0)
    def _():
        stats_ref[0, 0] = 0.0
        stats_ref[0, 1] = 0.0

    def produce(f_ref, buf, fresh):
        f = f_ref[...]
        ss = jnp.sum(f * f, axis=1, keepdims=True)
        fn = f * (1.0 / jnp.maximum(jnp.sqrt(ss), 1e-12))
        buf[...] = jnp.dot(fn, cbt_ref[...],
                           preferred_element_type=jnp.float32)
        stats_ref[0, 1] += jnp.where(fresh, jnp.sum(fn * fn), 0.0)

    def consume(buf, idx_out_ref, fresh):
        s = buf[...]
        m = jnp.max(s, axis=1, keepdims=True)
        mask = s >= m
        idx_out_ref[0, 0, :] = jnp.argmax(s, axis=1).astype(jnp.int32)
        stats_ref[0, 0] += jnp.where(fresh, jnp.sum(m), 0.0)
        return jnp.any(mask.reshape(_TOK_BLK // 8, 8, _VOCAB), axis=0)

    produce(f_a_ref, buf_a, j < m_steps)
    mf_b = consume(buf_b, idx_o_ref, j > 0)
    produce(f_b_ref, buf_b, j < m_steps)
    mf_a = consume(buf_a, idx_e_ref, j < m_steps)

    g_b = jnp.logical_and(j > 0, mf_b)
    contrib = jnp.logical_or(mf_a, g_b).astype(jnp.int32)
    used_ref[...] = jnp.where(j == 0, contrib, used_ref[...] | contrib)


def _tc_argmax(f, cbn_t):
    n = f.shape[0]
    nblk = n // _TOK_BLK
    m_steps = nblk // 2
    body = functools.partial(_argmax_body, nblk)
    return pl.pallas_call(
        body,
        grid=(m_steps + 1,),
        in_specs=[
            pl.BlockSpec((_TOK_BLK, _WIDTH),
                         lambda j: (jnp.minimum(2 * j, nblk - 2), 0)),
            pl.BlockSpec((_TOK_BLK, _WIDTH),
                         lambda j: (jnp.minimum(2 * j + 1, nblk - 1), 0)),
            pl.BlockSpec((_WIDTH, _VOCAB), lambda j: (0, 0)),
        ],
        out_specs=[
            pl.BlockSpec((1, 1, _TOK_BLK),
                         lambda j: (jnp.minimum(j, m_steps - 1), 0, 0)),
            pl.BlockSpec((1, 1, _TOK_BLK),
                         lambda j: (jnp.maximum(j - 1, 0), 0, 0)),
            pl.BlockSpec((8, _VOCAB), lambda j: (0, 0)),
            pl.BlockSpec(memory_space=pltpu.SMEM),
        ],
        out_shape=[
            jax.ShapeDtypeStruct((m_steps, 1, _TOK_BLK), jnp.int32),
            jax.ShapeDtypeStruct((m_steps, 1, _TOK_BLK), jnp.int32),
            jax.ShapeDtypeStruct((8, _VOCAB), jnp.int32),
            jax.ShapeDtypeStruct((1, 4), jnp.float32),
        ],
        scratch_shapes=[
            pltpu.VMEM((_TOK_BLK, _VOCAB), jnp.float32),
            pltpu.VMEM((_TOK_BLK, _VOCAB), jnp.float32),
        ],
    )(f, f, cbn_t)


def _prep(cb):
    return pl.pallas_call(
        _prep_body,
        out_shape=[
            jax.ShapeDtypeStruct((_VOCAB, _GATHER_PAD), jnp.float32),
            jax.ShapeDtypeStruct((_WIDTH, _VOCAB), jnp.float32),
        ],
    )(cb)


def _sc_gather(cbn, idx2d):
    n = idx2d.shape[1]
    mesh = plsc.VectorSubcoreMesh(core_axis_name="c", subcore_axis_name="s")

    @functools.partial(
        pl.kernel,
        out_type=jax.ShapeDtypeStruct((n, _GATHER_PAD), jnp.float32),
        mesh=mesh,
    )
    def gather_kernel(cb_hbm, i_hbm, o_hbm):
        def body(i_vmem, o_vmem):
            pltpu.sync_copy(cb_hbm.at[i_vmem.at[0]], o_vmem)

        pltpu.emit_pipeline(
            body,
            grid=(n // _GATHER_WIN,),
            in_specs=[pl.BlockSpec((1, _GATHER_WIN), lambda i: (0, i))],
            out_specs=[pl.BlockSpec((_GATHER_WIN, _GATHER_PAD),
                                    lambda i: (i, 0))],
            core_axis_name="s",
            dimension_semantics=(pltpu.PARALLEL,),
        )(i_hbm, o_hbm)

    return gather_kernel(cbn, idx2d)


def _shard_fn(n_total, axis, features, codebook_weight):
    b, l, c = features.shape
    f = features.reshape(-1, c)
    cbn, cbn_t = _prep(codebook_weight)
    idx_e, idx_o, used, stats = _tc_argmax(f, cbn_t)
    idx = jnp.stack([idx_e[:, 0, :], idx_o[:, 0, :]], axis=1)
    f_hat = _sc_gather(cbn, idx.reshape(1, -1))[:, :_WIDTH]
    s_sum = stats[0, 0]
    nf2_sum = stats[0, 1]
    if axis is not None:
        s_sum = jax.lax.psum(s_sum, axis)
        nf2_sum = jax.lax.psum(nf2_sum, axis)
        used = jax.lax.pmax(used, axis)
    n = jnp.float32(n_total)
    vq_loss = (1.0 + _BETA) * (nf2_sum - 2.0 * s_sum + n) / (n * _WIDTH)
    used_row = jnp.max(used, axis=0).astype(jnp.float32)
    vocab_usage = 100.0 * jnp.sum(used_row) / _VOCAB
    return (f_hat.reshape(b, l, c), vq_loss, jnp.float32(0.0), vocab_usage)


def kernel(features, codebook_weight):
    n_total = features.shape[0] * features.shape[1]
    return _shard_fn(n_total, None, features, codebook_weight)


# speedup vs baseline: 1.7429x; 1.1960x over previous
import dataclasses
import functools

import jax
import jax.numpy as jnp
from jax.experimental import pallas as pl
from jax.experimental.pallas import tpu as pltpu
from jax.experimental.pallas import tpu_sc as plsc

_VOCAB = 8192
_WIDTH = 32
_BETA = 0.25
_TOK_BLK = 256
_GATHER_WIN = 128
_GATHER_PAD = 128


def _prep_body(cb_ref, cbn_ref, cbt_ref):
    cb = cb_ref[...]
    ss = jnp.sum(cb * cb, axis=1, keepdims=True)
    r = 1.0 / jnp.maximum(jnp.sqrt(ss), 1e-12)
    cbn = cb * r
    pad = jnp.zeros((_VOCAB, _GATHER_PAD - _WIDTH), jnp.float32)
    cbn_ref[...] = jnp.concatenate([cbn, pad], axis=1)
    cbt_ref[...] = cbn.T


def _argmax_body(nblk, f_a_ref, f_b_ref, cbt_ref, idx_e_ref, idx_o_ref,
                 stats_ref, buf_a, buf_b):
    j = pl.program_id(0)
    m_steps = nblk // 2

    @pl.when(j == 0)
    def _():
        stats_ref[0, 0] = 0.0
        stats_ref[0, 1] = 0.0

    def produce(f_ref, buf, fresh):
        f = f_ref[...]
        ss = jnp.sum(f * f, axis=1, keepdims=True)
        fn = f * (1.0 / jnp.maximum(jnp.sqrt(ss), 1e-12))
        buf[...] = jnp.dot(fn, cbt_ref[...],
                           preferred_element_type=jnp.float32)
        stats_ref[0, 1] += jnp.where(fresh, jnp.sum(fn * fn), 0.0)

    def consume(buf, idx_out_ref, fresh):
        s = buf[...]
        m = jnp.max(s, axis=1, keepdims=True)
        iota = jax.lax.broadcasted_iota(jnp.int32, s.shape, 1)
        idx_out_ref[0, 0, :] = jnp.min(jnp.where(s >= m, iota, _VOCAB),
                                       axis=1)
        stats_ref[0, 0] += jnp.where(fresh, jnp.sum(m), 0.0)

    produce(f_a_ref, buf_a, j < m_steps)
    consume(buf_b, idx_o_ref, j > 0)
    produce(f_b_ref, buf_b, j < m_steps)
    consume(buf_a, idx_e_ref, j < m_steps)


def _tc_argmax(f, cbn_t):
    n = f.shape[0]
    nblk = n // _TOK_BLK
    m_steps = nblk // 2
    body = functools.partial(_argmax_body, nblk)
    return pl.pallas_call(
        body,
        grid=(m_steps + 1,),
        in_specs=[
            pl.BlockSpec((_TOK_BLK, _WIDTH),
                         lambda j: (jnp.minimum(2 * j, nblk - 2), 0)),
            pl.BlockSpec((_TOK_BLK, _WIDTH),
                         lambda j: (jnp.minimum(2 * j + 1, nblk - 1), 0)),
            pl.BlockSpec((_WIDTH, _VOCAB), lambda j: (0, 0)),
        ],
        out_specs=[
            pl.BlockSpec((1, 1, _TOK_BLK),
                         lambda j: (jnp.minimum(j, m_steps - 1), 0, 0)),
            pl.BlockSpec((1, 1, _TOK_BLK),
                         lambda j: (jnp.maximum(j - 1, 0), 0, 0)),
            pl.BlockSpec(memory_space=pltpu.SMEM),
        ],
        out_shape=[
            jax.ShapeDtypeStruct((m_steps, 1, _TOK_BLK), jnp.int32),
            jax.ShapeDtypeStruct((m_steps, 1, _TOK_BLK), jnp.int32),
            jax.ShapeDtypeStruct((1, 4), jnp.float32),
        ],
        scratch_shapes=[
            pltpu.VMEM((_TOK_BLK, _VOCAB), jnp.float32),
            pltpu.VMEM((_TOK_BLK, _VOCAB), jnp.float32),
        ],
    )(f, f, cbn_t)


def _prep(cb):
    return pl.pallas_call(
        _prep_body,
        out_shape=[
            jax.ShapeDtypeStruct((_VOCAB, _GATHER_PAD), jnp.float32),
            jax.ShapeDtypeStruct((_WIDTH, _VOCAB), jnp.float32),
        ],
    )(cb)


def _sc_gather(cbn, idx2d):
    n = idx2d.shape[1]
    mesh = plsc.VectorSubcoreMesh(core_axis_name="c", subcore_axis_name="s")
    n_sub = 16
    cp = pltpu.CompilerParams()
    if "needs_layout_passes" in pltpu.CompilerParams.__dataclass_fields__:
        cp = dataclasses.replace(cp, needs_layout_passes=False)

    @functools.partial(
        pl.kernel,
        out_type=(
            jax.ShapeDtypeStruct((n, _GATHER_PAD), jnp.float32),
            jax.ShapeDtypeStruct((n_sub, _VOCAB), jnp.int32),
        ),
        mesh=mesh,
        scratch_types=[pltpu.VMEM((_VOCAB,), jnp.int32)],
        compiler_params=cp,
    )
    def gather_kernel(cb_hbm, i_hbm, o_hbm, u_hbm, table):
        @pl.loop(0, _VOCAB, step=16)
        def _(k):
            table[pl.ds(k, 16)] = jnp.zeros((16,), jnp.int32)

        ones = jnp.ones((16,), jnp.int32)

        def body(i_vmem, o_vmem):
            pltpu.sync_copy(cb_hbm.at[i_vmem.at[0]], o_vmem)
            for k in range(_GATHER_WIN // 16):
                iv = i_vmem[0, pl.ds(k * 16, 16)]
                plsc.store_scatter(table, [iv], ones)

        pltpu.emit_pipeline(
            body,
            grid=(n // _GATHER_WIN,),
            in_specs=[pl.BlockSpec((1, _GATHER_WIN), lambda i: (0, i))],
            out_specs=[pl.BlockSpec((_GATHER_WIN, _GATHER_PAD),
                                    lambda i: (i, 0))],
            core_axis_name="s",
            dimension_semantics=(pltpu.PARALLEL,),
        )(i_hbm, o_hbm)

        sid = jax.lax.axis_index("s")
        pltpu.sync_copy(table, u_hbm.at[sid])

    return gather_kernel(cbn, idx2d)


def _count_body(u_ref, out_ref):
    used_any = jnp.max(u_ref[...], axis=0).astype(jnp.float32)
    out_ref[0, 0] = 100.0 * jnp.sum(used_any) / _VOCAB


def _count_used(u):
    return pl.pallas_call(
        _count_body,
        out_specs=pl.BlockSpec(memory_space=pltpu.SMEM),
        out_shape=jax.ShapeDtypeStruct((1, 1), jnp.float32),
    )(u)


def kernel(features, codebook_weight):
    b, l, c = features.shape
    f = features.reshape(-1, c)
    cbn, cbn_t = _prep(codebook_weight)
    idx_e, idx_o, stats = _tc_argmax(f, cbn_t)
    idx = jnp.stack([idx_e[:, 0, :], idx_o[:, 0, :]], axis=1)
    f_hat, used = _sc_gather(cbn, idx.reshape(1, -1))
    f_hat = f_hat[:, :_WIDTH]
    n = jnp.float32(f.shape[0])
    s_sum = stats[0, 0]
    nf2_sum = stats[0, 1]
    vq_loss = (1.0 + _BETA) * (nf2_sum - 2.0 * s_sum + n) / (n * _WIDTH)
    vocab_usage = _count_used(used)[0, 0]
    return (f_hat.reshape(b, l, c), vq_loss, jnp.float32(0.0), vocab_usage)


# speedup vs baseline: 1.8694x; 1.0725x over previous
import dataclasses
import functools

import jax
import jax.numpy as jnp
from jax.experimental import pallas as pl
from jax.experimental.pallas import tpu as pltpu
from jax.experimental.pallas import tpu_sc as plsc

_VOCAB = 8192
_WIDTH = 32
_BETA = 0.25
_TOK_BLK = 512
_GATHER_WIN = 128
_GATHER_PAD = 128


def _prep_body(cb_ref, cbn_ref, cbt_ref):
    cb = cb_ref[...]
    ss = jnp.sum(cb * cb, axis=1, keepdims=True)
    r = 1.0 / jnp.maximum(jnp.sqrt(ss), 1e-12)
    cbn = cb * r
    pad = jnp.zeros((_VOCAB, _GATHER_PAD - _WIDTH), jnp.float32)
    cbn_ref[...] = jnp.concatenate([cbn, pad], axis=1)
    cbt_ref[...] = cbn.T


def _argmax_body(nblk, f_a_ref, f_b_ref, cbt_ref, idx_e_ref, idx_o_ref,
                 stats_ref, buf_a, buf_b):
    j = pl.program_id(0)
    m_steps = nblk // 2

    @pl.when(j == 0)
    def _():
        stats_ref[0, 0] = 0.0
        stats_ref[0, 1] = 0.0

    def produce(f_ref, buf, fresh):
        f = f_ref[...]
        ss = jnp.sum(f * f, axis=1, keepdims=True)
        fn = f * (1.0 / jnp.maximum(jnp.sqrt(ss), 1e-12))
        buf[...] = jnp.dot(fn, cbt_ref[...],
                           preferred_element_type=jnp.float32)
        stats_ref[0, 1] += jnp.where(fresh, jnp.sum(fn * fn), 0.0)

    def consume(buf, idx_out_ref, fresh):
        s = buf[...]
        m = jnp.max(s, axis=1, keepdims=True)
        iota = jax.lax.broadcasted_iota(jnp.int32, s.shape, 1)
        idx_out_ref[0, 0, :] = jnp.min(jnp.where(s >= m, iota, _VOCAB),
                                       axis=1)
        stats_ref[0, 0] += jnp.where(fresh, jnp.sum(m), 0.0)

    produce(f_a_ref, buf_a, j < m_steps)
    consume(buf_b, idx_o_ref, j > 0)
    produce(f_b_ref, buf_b, j < m_steps)
    consume(buf_a, idx_e_ref, j < m_steps)


def _tc_argmax(f, cbn_t):
    n = f.shape[0]
    nblk = n // _TOK_BLK
    m_steps = nblk // 2
    body = functools.partial(_argmax_body, nblk)
    return pl.pallas_call(
        body,
        grid=(m_steps + 1,),
        in_specs=[
            pl.BlockSpec((_TOK_BLK, _WIDTH),
                         lambda j: (jnp.minimum(2 * j, nblk - 2), 0)),
            pl.BlockSpec((_TOK_BLK, _WIDTH),
                         lambda j: (jnp.minimum(2 * j + 1, nblk - 1), 0)),
            pl.BlockSpec((_WIDTH, _VOCAB), lambda j: (0, 0)),
        ],
        out_specs=[
            pl.BlockSpec((1, 1, _TOK_BLK),
                         lambda j: (jnp.minimum(j, m_steps - 1), 0, 0)),
            pl.BlockSpec((1, 1, _TOK_BLK),
                         lambda j: (jnp.maximum(j - 1, 0), 0, 0)),
            pl.BlockSpec(memory_space=pltpu.SMEM),
        ],
        out_shape=[
            jax.ShapeDtypeStruct((m_steps, 1, _TOK_BLK), jnp.int32),
            jax.ShapeDtypeStruct((m_steps, 1, _TOK_BLK), jnp.int32),
            jax.ShapeDtypeStruct((1, 4), jnp.float32),
        ],
        scratch_shapes=[
            pltpu.VMEM((_TOK_BLK, _VOCAB), jnp.float32),
            pltpu.VMEM((_TOK_BLK, _VOCAB), jnp.float32),
        ],
    )(f, f, cbn_t)


def _prep(cb):
    return pl.pallas_call(
        _prep_body,
        out_shape=[
            jax.ShapeDtypeStruct((_VOCAB, _GATHER_PAD), jnp.float32),
            jax.ShapeDtypeStruct((_WIDTH, _VOCAB), jnp.float32),
        ],
    )(cb)


def _sc_gather(cbn, idx2d):
    n = idx2d.shape[1]
    mesh = plsc.VectorSubcoreMesh(core_axis_name="c", subcore_axis_name="s")
    n_sub = 16
    cp = pltpu.CompilerParams()
    if "needs_layout_passes" in pltpu.CompilerParams.__dataclass_fields__:
        cp = dataclasses.replace(cp, needs_layout_passes=False)

    @functools.partial(
        pl.kernel,
        out_type=(
            jax.ShapeDtypeStruct((n, _GATHER_PAD), jnp.float32),
            jax.ShapeDtypeStruct((n_sub, _VOCAB), jnp.int32),
        ),
        mesh=mesh,
        scratch_types=[pltpu.VMEM((_VOCAB,), jnp.int32)],
        compiler_params=cp,
    )
    def gather_kernel(cb_hbm, i_hbm, o_hbm, u_hbm, table):
        @pl.loop(0, _VOCAB, step=16)
        def _(k):
            table[pl.ds(k, 16)] = jnp.zeros((16,), jnp.int32)

        ones = jnp.ones((16,), jnp.int32)

        def body(i_vmem, o_vmem):
            pltpu.sync_copy(cb_hbm.at[i_vmem.at[0]], o_vmem)
            for k in range(_GATHER_WIN // 16):
                iv = i_vmem[0, pl.ds(k * 16, 16)]
                plsc.store_scatter(table, [iv], ones)

        pltpu.emit_pipeline(
            body,
            grid=(n // _GATHER_WIN,),
            in_specs=[pl.BlockSpec((1, _GATHER_WIN), lambda i: (0, i))],
            out_specs=[pl.BlockSpec((_GATHER_WIN, _GATHER_PAD),
                                    lambda i: (i, 0))],
            core_axis_name="s",
            dimension_semantics=(pltpu.PARALLEL,),
        )(i_hbm, o_hbm)

        sid = jax.lax.axis_index("s")
        pltpu.sync_copy(table, u_hbm.at[sid])

    return gather_kernel(cbn, idx2d)


def _count_body(u_ref, out_ref):
    used_any = jnp.max(u_ref[...], axis=0).astype(jnp.float32)
    out_ref[0, 0] = 100.0 * jnp.sum(used_any) / _VOCAB


def _count_used(u):
    return pl.pallas_call(
        _count_body,
        out_specs=pl.BlockSpec(memory_space=pltpu.SMEM),
        out_shape=jax.ShapeDtypeStruct((1, 1), jnp.float32),
    )(u)


def kernel(features, codebook_weight):
    b, l, c = features.shape
    f = features.reshape(-1, c)
    cbn, cbn_t = _prep(codebook_weight)
    idx_e, idx_o, stats = _tc_argmax(f, cbn_t)
    idx = jnp.stack([idx_e[:, 0, :], idx_o[:, 0, :]], axis=1)
    f_hat, used = _sc_gather(cbn, idx.reshape(1, -1))
    f_hat = f_hat[:, :_WIDTH]
    n = jnp.float32(f.shape[0])
    s_sum = stats[0, 0]
    nf2_sum = stats[0, 1]
    vq_loss = (1.0 + _BETA) * (nf2_sum - 2.0 * s_sum + n) / (n * _WIDTH)
    vocab_usage = _count_used(used)[0, 0]
    return (f_hat.reshape(b, l, c), vq_loss, jnp.float32(0.0), vocab_usage)
